# baseline (speedup 1.0000x reference)
import functools

import jax
import jax.numpy as jnp
from jax import lax
from jax.experimental import pallas as pl
from jax.experimental.pallas import tpu as pltpu
from jax.experimental.pallas import tpu_sc as plsc

NUM_EMB = 100000
EMB_DIM = 64
HN = 3200000
ROWS = HN // EMB_DIM
BATCH = 4096
BAG = 20

NW = 32
BAGS_PER_W = BATCH // NW
CB = 8
CHUNK_IDX = CB * BAG
CHUNKS = BAGS_PER_W // CB
LANES = 16
COLS = EMB_DIM // LANES
NBUF = 2
PACK = EMB_DIM // 2


def _bag_kernel(w2, idx):
    mesh = plsc.VectorSubcoreMesh(core_axis_name="c", subcore_axis_name="s")

    @functools.partial(
        pl.kernel,
        mesh=mesh,
        compiler_params=pltpu.CompilerParams(
            use_tc_tiling_on_sc=False,
            skip_device_barrier=True,
            disable_bounds_checks=True,
            disable_semaphore_checks=True,
        ),
        out_type=jax.ShapeDtypeStruct((BATCH, EMB_DIM), jnp.float32),
        scratch_types=[
            pltpu.VMEM((CHUNKS, CHUNK_IDX), jnp.int32),
        ] + [pltpu.VMEM((CHUNK_IDX, EMB_DIM), jnp.float32)] * NBUF + [
            pltpu.VMEM((BAGS_PER_W, EMB_DIM), jnp.float32),
        ] + [pltpu.SemaphoreType.DMA] * NBUF,
    )
    def k(w2_hbm, idx_hbm, out_hbm, idx_v, *rest):
        rows = rest[:NBUF]
        out_v = rest[NBUF]
        sems = rest[NBUF + 1:]
        wid = lax.axis_index("s") * 2 + lax.axis_index("c")
        pltpu.sync_copy(idx_hbm.at[wid], idx_v)

        @pl.loop(0, CHUNKS)
        def _(c):
            for k5 in range(CHUNK_IDX // LANES):
                sl = pl.ds(k5 * LANES, LANES)
                v = idx_v[c, sl]
                idx_v[c, sl] = jnp.where(v >= ROWS, v - ROWS, v)

        def start(c, buf, sem):
            pltpu.async_copy(w2_hbm.at[idx_v.at[c]], buf, sem)

        def wait(c, buf, sem):
            pltpu.make_async_copy(w2_hbm.at[idx_v.at[c]], buf, sem).wait()

        def accum(c, buf):
            for b in range(CB):
                for g in range(COLS):
                    sl = pl.ds(g * LANES, LANES)
                    vals = [buf[b * BAG + r, sl] for r in range(BAG)]
                    while len(vals) > 1:
                        nxt = [vals[i] + vals[i + 1]
                               for i in range(0, len(vals) - 1, 2)]
                        if len(vals) % 2:
                            nxt.append(vals[-1])
                        vals = nxt
                    out_v[c * CB + b, sl] = vals[0]

        for j in range(NBUF - 1):
            start(j, rows[j], sems[j])

        @pl.loop(0, CHUNKS - NBUF, step=NBUF)
        def _(c):
            for j in range(NBUF):
                start(c + j + NBUF - 1, rows[(j - 1) % NBUF], sems[(j - 1) % NBUF])
                wait(c + j, rows[j], sems[j])
                accum(c + j, rows[j])

        cl = CHUNKS - NBUF
        start(CHUNKS - 1, rows[(CHUNKS - 1) % NBUF], sems[(CHUNKS - 1) % NBUF])
        for j in range(NBUF):
            wait(cl + j, rows[j], sems[j])
            accum(cl + j, rows[j])

        pltpu.sync_copy(out_v, out_hbm.at[pl.ds(wid * BAGS_PER_W, BAGS_PER_W)])

    return k(w2, idx)


def kernel(x, hashed_weight):
    w2 = hashed_weight.reshape(ROWS, EMB_DIM)
    idx = x.reshape(NW, CHUNKS, CHUNK_IDX)
    return _bag_kernel(w2, idx)


# speedup vs baseline: 1.2233x; 1.2233x over previous
import functools

import jax
import jax.numpy as jnp
from jax import lax
from jax.experimental import pallas as pl
from jax.experimental.pallas import tpu as pltpu
from jax.experimental.pallas import tpu_sc as plsc

NUM_EMB = 100000
EMB_DIM = 64
HN = 3200000
ROWS = HN // EMB_DIM
BATCH = 4096
BAG = 20

NW = 32
BAGS_PER_W = BATCH // NW
CB = 4
CHUNK_IDX = CB * BAG
CHUNKS = BAGS_PER_W // CB
LANES = 16
COLS = EMB_DIM // LANES
NBUF = 2
PACK = EMB_DIM // 2


def _bag_kernel(w2, idx):
    mesh = plsc.VectorSubcoreMesh(core_axis_name="c", subcore_axis_name="s")

    @functools.partial(
        pl.kernel,
        mesh=mesh,
        compiler_params=pltpu.CompilerParams(
            use_tc_tiling_on_sc=False,
            skip_device_barrier=True,
            disable_bounds_checks=True,
            disable_semaphore_checks=True,
        ),
        out_type=jax.ShapeDtypeStruct((BATCH, EMB_DIM), jnp.float32),
        scratch_types=[
            pltpu.VMEM((CHUNKS, CHUNK_IDX), jnp.int32),
        ] + [pltpu.VMEM((CHUNK_IDX, EMB_DIM), jnp.float32)] * NBUF + [
            pltpu.VMEM((BAGS_PER_W, EMB_DIM), jnp.float32),
        ] + [pltpu.SemaphoreType.DMA] * NBUF,
    )
    def k(w2_hbm, idx_hbm, out_hbm, idx_v, *rest):
        rows = rest[:NBUF]
        out_v = rest[NBUF]
        sems = rest[NBUF + 1:]
        wid = lax.axis_index("s") * 2 + lax.axis_index("c")
        pltpu.sync_copy(idx_hbm.at[wid], idx_v)

        def start(c, buf, sem):
            for k5 in range(CHUNK_IDX // LANES):
                sl = pl.ds(k5 * LANES, LANES)
                v = idx_v[c, sl]
                idx_v[c, sl] = jnp.where(v >= ROWS, v - ROWS, v)
            pltpu.async_copy(w2_hbm.at[idx_v.at[c]], buf, sem)

        def wait(c, buf, sem):
            pltpu.make_async_copy(w2_hbm.at[idx_v.at[c]], buf, sem).wait()

        def accum(c, buf):
            for b in range(CB):
                for g in range(COLS):
                    sl = pl.ds(g * LANES, LANES)
                    vals = [buf[b * BAG + r, sl] for r in range(BAG)]
                    while len(vals) > 1:
                        nxt = [vals[i] + vals[i + 1]
                               for i in range(0, len(vals) - 1, 2)]
                        if len(vals) % 2:
                            nxt.append(vals[-1])
                        vals = nxt
                    out_v[c * CB + b, sl] = vals[0]

        for j in range(NBUF - 1):
            start(j, rows[j], sems[j])

        @pl.loop(0, CHUNKS - NBUF, step=NBUF)
        def _(c):
            for j in range(NBUF):
                start(c + j + NBUF - 1, rows[(j - 1) % NBUF], sems[(j - 1) % NBUF])
                wait(c + j, rows[j], sems[j])
                accum(c + j, rows[j])

        cl = CHUNKS - NBUF
        start(CHUNKS - 1, rows[(CHUNKS - 1) % NBUF], sems[(CHUNKS - 1) % NBUF])
        for j in range(NBUF):
            wait(cl + j, rows[j], sems[j])
            accum(cl + j, rows[j])

        pltpu.sync_copy(out_v, out_hbm.at[pl.ds(wid * BAGS_PER_W, BAGS_PER_W)])

    return k(w2, idx)


def kernel(x, hashed_weight):
    w2 = hashed_weight.reshape(ROWS, EMB_DIM)
    idx = x.reshape(NW, CHUNKS, CHUNK_IDX)
    return _bag_kernel(w2, idx)


# speedup vs baseline: 1.2713x; 1.0392x over previous
import functools

import jax
import jax.numpy as jnp
from jax import lax
from jax.experimental import pallas as pl
from jax.experimental.pallas import tpu as pltpu
from jax.experimental.pallas import tpu_sc as plsc

NUM_EMB = 100000
EMB_DIM = 64
HN = 3200000
ROWS = HN // EMB_DIM
BATCH = 4096
BAG = 20

NW = 32
BAGS_PER_W = BATCH // NW
CB = 4
CHUNK_IDX = CB * BAG
CHUNKS = BAGS_PER_W // CB
LANES = 16
COLS = EMB_DIM // LANES
NBUF = 2
PACK = EMB_DIM // 2


def _bag_kernel(w2, idx):
    mesh = plsc.VectorSubcoreMesh(core_axis_name="c", subcore_axis_name="s")

    @functools.partial(
        pl.kernel,
        mesh=mesh,
        compiler_params=pltpu.CompilerParams(
            use_tc_tiling_on_sc=False,
            skip_device_barrier=True,
            disable_bounds_checks=True,
            disable_semaphore_checks=True,
        ),
        out_type=jax.ShapeDtypeStruct((BATCH, EMB_DIM), jnp.float32),
        scratch_types=[
            pltpu.VMEM((CHUNKS, CHUNK_IDX), jnp.int32),
        ] + [pltpu.VMEM((CHUNK_IDX, EMB_DIM), jnp.float32)] * NBUF + [
            pltpu.VMEM((BAGS_PER_W, EMB_DIM), jnp.float32),
        ] + [pltpu.SemaphoreType.DMA] * NBUF,
    )
    def k(w2_hbm, idx_hbm, out_hbm, idx_v, *rest):
        rows = rest[:NBUF]
        out_v = rest[NBUF]
        sems = rest[NBUF + 1:]
        wid = lax.axis_index("s") * 2 + lax.axis_index("c")
        pltpu.sync_copy(idx_hbm.at[wid], idx_v)

        def start(c, buf, sem):
            for k5 in range(CHUNK_IDX // LANES):
                sl = pl.ds(k5 * LANES, LANES)
                v = idx_v[c, sl]
                idx_v[c, sl] = jnp.where(v >= ROWS, v - ROWS, v)
            pltpu.async_copy(w2_hbm.at[idx_v.at[c]], buf, sem)

        def wait(c, buf, sem):
            pltpu.make_async_copy(w2_hbm.at[idx_v.at[c]], buf, sem).wait()

        def accum(c, buf):
            for b in range(CB):
                sls = [pl.ds(g * LANES, LANES) for g in range(COLS)]
                accs = [buf[b * BAG, sl] for sl in sls]
                for r in range(1, BAG):
                    for g in range(COLS):
                        accs[g] = accs[g] + buf[b * BAG + r, sls[g]]
                for g in range(COLS):
                    out_v[c * CB + b, sls[g]] = accs[g]

        for j in range(NBUF - 1):
            start(j, rows[j], sems[j])

        @pl.loop(0, CHUNKS - NBUF, step=NBUF)
        def _(c):
            for j in range(NBUF):
                start(c + j + NBUF - 1, rows[(j - 1) % NBUF], sems[(j - 1) % NBUF])
                wait(c + j, rows[j], sems[j])
                accum(c + j, rows[j])

        cl = CHUNKS - NBUF
        start(CHUNKS - 1, rows[(CHUNKS - 1) % NBUF], sems[(CHUNKS - 1) % NBUF])
        for j in range(NBUF):
            wait(cl + j, rows[j], sems[j])
            accum(cl + j, rows[j])

        pltpu.sync_copy(out_v, out_hbm.at[pl.ds(wid * BAGS_PER_W, BAGS_PER_W)])

    return k(w2, idx)


def kernel(x, hashed_weight):
    w2 = hashed_weight.reshape(ROWS, EMB_DIM)
    idx = x.reshape(NW, CHUNKS, CHUNK_IDX)
    return _bag_kernel(w2, idx)
